# baseline (speedup 1.0000x reference)
import jax
import jax.numpy as jnp
from jax.experimental import pallas as pl
from jax.experimental.pallas import tpu as pltpu

_NDIM = 3
_EPS = 1e-5
_CHUNK = 512
_S_IN = 8


def _leaky(x):
    return jnp.where(x >= 0, x, 0.01 * x)


def _bn(x, g, b):
    mu = jnp.mean(x, axis=0, keepdims=True)
    var = jnp.mean((x - mu) ** 2, axis=0, keepdims=True)
    return g * ((x - mu) * jax.lax.rsqrt(var + _EPS)) + b


def _dense(x, w, b):
    return jnp.dot(x, w, preferred_element_type=jnp.float32) + b


def _forward_kernel(w_in_ref, pos_ref, *refs):
    params = [r[...] for r in refs[:26]]
    pos_out, w_out = refs[26], refs[27]
    x0_scr, h_scr, dp_scr = refs[28], refs[29], refs[30]

    s = pl.program_id(0)

    it = iter(params)

    def take(n):
        return [next(it) for _ in range(n)]

    riW0, riB0, riG0, riBt0, riW1, riB1, riG1, riBt1 = take(8)
    blk0 = take(6)
    blk1 = take(6)
    roW0, roB0, roG0, roBt0, roW1, roB1 = take(6)

    @pl.when(s < _S_IN)
    def _stream_in():
        x0_scr[pl.ds(s * _CHUNK, _CHUNK), :] = jnp.dot(
            w_in_ref[...], riW0, preferred_element_type=jnp.float32)

    @pl.when(s == _S_IN)
    def _middle():
        x = _leaky(_bn(x0_scr[...] + riB0, riG0, riBt0))
        w = _leaky(_bn(_dense(x, riW1, riB1), riG1, riBt1))
        dp = jnp.zeros((x.shape[0], _NDIM), jnp.float32)
        for dW0, dB0, dG0, dBt0, dW1, dB1 in (blk0, blk1):
            h = _leaky(_bn(_dense(w, dW0, dB0), dG0, dBt0))
            dp = dp + _dense(h, dW1[:, :_NDIM], dB1[:_NDIM])
            w = w + _dense(h, dW1[:, _NDIM:], dB1[_NDIM:])
        h_scr[...] = _leaky(_bn(_dense(w, roW0, roB0), roG0, roBt0))
        dp_scr[...] = dp

    @pl.when(s >= _S_IN)
    def _stream_out():
        c = s - _S_IN
        hc = h_scr[pl.ds(c * _CHUNK, _CHUNK), :]
        w_out[...] = _dense(hc, roW1, roB1)
        pos_out[...] = pos_ref[...] + dp_scr[pl.ds(c * _CHUNK, _CHUNK), :]


def kernel(positions, weights, params, batch):
    del batch
    n = positions.shape[0]

    flat = []
    for p in params["readin"]:
        flat += [p["W"], p["b"], p["gamma"], p["beta"]]
    for blk in params["blocks"]:
        l0, l1 = blk["delta"]
        flat += [l0["W"], l0["b"], l0["gamma"], l0["beta"], l1["W"], l1["b"]]
    ro0, ro1 = params["readout"]
    flat += [ro0["W"], ro0["b"], ro0["gamma"], ro0["beta"], ro1["W"], ro1["b"]]

    hmlp = ro0["W"].shape[1]
    out_dim = ro1["W"].shape[1]
    n_in = _S_IN
    grid = (2 * n_in,)

    def _const(shape):
        zeros = (0,) * len(shape)
        return pl.BlockSpec(shape, lambda s, _z=zeros: _z)

    in_specs = [
        pl.BlockSpec((_CHUNK, weights.shape[1]),
                     lambda s: (jnp.minimum(s, n_in - 1), 0)),
        pl.BlockSpec((_CHUNK, _NDIM),
                     lambda s: (jnp.maximum(s - n_in, 0), 0)),
    ] + [_const(p.shape) for p in flat]

    pos_out, w_out = pl.pallas_call(
        _forward_kernel,
        grid=grid,
        in_specs=in_specs,
        out_specs=(
            pl.BlockSpec((_CHUNK, _NDIM), lambda s: (jnp.maximum(s - n_in, 0), 0)),
            pl.BlockSpec((_CHUNK, out_dim), lambda s: (jnp.maximum(s - n_in, 0), 0)),
        ),
        out_shape=(
            jax.ShapeDtypeStruct((n, _NDIM), jnp.float32),
            jax.ShapeDtypeStruct((n, out_dim), jnp.float32),
        ),
        scratch_shapes=[
            pltpu.VMEM((n, hmlp), jnp.float32),
            pltpu.VMEM((n, hmlp), jnp.float32),
            pltpu.VMEM((n, _NDIM), jnp.float32),
        ],
    )(weights, positions, *flat)
    return pos_out, w_out


# speedup vs baseline: 1.1420x; 1.1420x over previous
import jax
import jax.numpy as jnp
from jax.experimental import pallas as pl
from jax.experimental.pallas import tpu as pltpu

_NDIM = 3
_EPS = 1e-5
_CHUNK = 1024
_S_IN = 4


def _leaky(x):
    return jnp.where(x >= 0, x, 0.01 * x)


def _bn(x, g, b):
    mu = jnp.mean(x, axis=0, keepdims=True)
    var = jnp.mean((x - mu) ** 2, axis=0, keepdims=True)
    return g * ((x - mu) * jax.lax.rsqrt(var + _EPS)) + b


def _dense(x, w, b):
    return jnp.dot(x, w, preferred_element_type=jnp.float32) + b


def _forward_kernel(w_in_ref, pos_ref, *refs):
    prefs = refs[:26]
    pos_out, w_out = refs[26], refs[27]
    x0_scr, h_scr, dp_scr = refs[28], refs[29], refs[30]

    s = pl.program_id(0)

    @pl.when(s < _S_IN)
    def _stream_in():
        x0_scr[pl.ds(s * _CHUNK, _CHUNK), :] = jnp.dot(
            w_in_ref[...], prefs[0][...], preferred_element_type=jnp.float32)

    @pl.when(s == _S_IN)
    def _middle():
        it = iter(prefs)

        def take(n):
            return [next(it)[...] for n_ in range(n)]

        riW0_unused, riB0, riG0, riBt0, riW1, riB1, riG1, riBt1 = take(8)
        blk0 = take(6)
        blk1 = take(6)
        roW0, roB0, roG0, roBt0 = take(4)

        x = _leaky(_bn(x0_scr[...] + riB0, riG0, riBt0))
        w = _leaky(_bn(_dense(x, riW1, riB1), riG1, riBt1))
        dp = jnp.zeros((x.shape[0], _NDIM), jnp.float32)
        for dW0, dB0, dG0, dBt0, dW1, dB1 in (blk0, blk1):
            h = _leaky(_bn(_dense(w, dW0, dB0), dG0, dBt0))
            dp = dp + _dense(h, dW1[:, :_NDIM], dB1[:_NDIM])
            w = w + _dense(h, dW1[:, _NDIM:], dB1[_NDIM:])
        h_scr[...] = _leaky(_bn(_dense(w, roW0, roB0), roG0, roBt0))
        dp_scr[...] = dp

    @pl.when(s >= _S_IN)
    def _stream_out():
        c = s - _S_IN
        hc = h_scr[pl.ds(c * _CHUNK, _CHUNK), :]
        w_out[...] = _dense(hc, prefs[24][...], prefs[25][...])
        pos_out[...] = pos_ref[...] + dp_scr[pl.ds(c * _CHUNK, _CHUNK), :]


def kernel(positions, weights, params, batch):
    del batch
    n = positions.shape[0]

    flat = []
    for p in params["readin"]:
        flat += [p["W"], p["b"], p["gamma"], p["beta"]]
    for blk in params["blocks"]:
        l0, l1 = blk["delta"]
        flat += [l0["W"], l0["b"], l0["gamma"], l0["beta"], l1["W"], l1["b"]]
    ro0, ro1 = params["readout"]
    flat += [ro0["W"], ro0["b"], ro0["gamma"], ro0["beta"], ro1["W"], ro1["b"]]

    hmlp = ro0["W"].shape[1]
    out_dim = ro1["W"].shape[1]
    n_in = _S_IN
    grid = (2 * n_in,)

    def _const(shape):
        zeros = (0,) * len(shape)
        return pl.BlockSpec(shape, lambda s, _z=zeros: _z)

    in_specs = [
        pl.BlockSpec((_CHUNK, weights.shape[1]),
                     lambda s: (jnp.minimum(s, n_in - 1), 0)),
        pl.BlockSpec((_CHUNK, _NDIM),
                     lambda s: (jnp.maximum(s - n_in, 0), 0)),
    ] + [_const(p.shape) for p in flat]

    pos_out, w_out = pl.pallas_call(
        _forward_kernel,
        grid=grid,
        in_specs=in_specs,
        out_specs=(
            pl.BlockSpec((_CHUNK, _NDIM), lambda s: (jnp.maximum(s - n_in, 0), 0)),
            pl.BlockSpec((_CHUNK, out_dim), lambda s: (jnp.maximum(s - n_in, 0), 0)),
        ),
        out_shape=(
            jax.ShapeDtypeStruct((n, _NDIM), jnp.float32),
            jax.ShapeDtypeStruct((n, out_dim), jnp.float32),
        ),
        scratch_shapes=[
            pltpu.VMEM((n, hmlp), jnp.float32),
            pltpu.VMEM((n, hmlp), jnp.float32),
            pltpu.VMEM((n, _NDIM), jnp.float32),
        ],
    )(weights, positions, *flat)
    return pos_out, w_out


# speedup vs baseline: 1.2701x; 1.1122x over previous
import jax
import jax.numpy as jnp
from jax.experimental import pallas as pl
from jax.experimental.pallas import tpu as pltpu

_NDIM = 3
_EPS = 1e-5
_N_PARAMS = 26


def _leaky(x):
    return jnp.where(x >= 0, x, 0.01 * x)


def _bn(x, g, b):
    mu = jnp.mean(x, axis=0, keepdims=True)
    var = jnp.mean((x - mu) ** 2, axis=0, keepdims=True)
    return g * ((x - mu) * jax.lax.rsqrt(var + _EPS)) + b


def _dense(x, w, b):
    return jnp.dot(x, w, preferred_element_type=jnp.float32) + b


def _forward_kernel(*refs):
    n_in = 2 + _N_PARAMS
    in_hbm = refs[:n_in]
    pos_out_hbm, w_out_hbm = refs[n_in], refs[n_in + 1]
    bufs = refs[n_in + 2:n_in + 2 + n_in]
    op_buf, ow_buf = refs[n_in + 2 + n_in], refs[n_in + 3 + n_in]
    sems = refs[n_in + 4 + n_in]
    out_sems = refs[n_in + 5 + n_in]

    copies = [
        pltpu.make_async_copy(in_hbm[i], bufs[i], sems.at[i])
        for i in range(n_in)
    ]
    for c in copies:
        c.start()
    for c in copies:
        c.wait()

    it = iter(range(2, n_in))

    def take(n):
        return [bufs[next(it)][...] for _ in range(n)]

    riW0, riB0, riG0, riBt0, riW1, riB1, riG1, riBt1 = take(8)
    blk0 = take(6)
    blk1 = take(6)
    roW0, roB0, roG0, roBt0, roW1, roB1 = take(6)

    x = _leaky(_bn(_dense(bufs[0][...], riW0, riB0), riG0, riBt0))
    w = _leaky(_bn(_dense(x, riW1, riB1), riG1, riBt1))
    dp = jnp.zeros((x.shape[0], _NDIM), jnp.float32)
    for dW0, dB0, dG0, dBt0, dW1, dB1 in (blk0, blk1):
        h = _leaky(_bn(_dense(w, dW0, dB0), dG0, dBt0))
        dp = dp + _dense(h, dW1[:, :_NDIM], dB1[:_NDIM])
        w = w + _dense(h, dW1[:, _NDIM:], dB1[_NDIM:])

    op_buf[...] = bufs[1][...] + dp
    pos_copy = pltpu.make_async_copy(op_buf, pos_out_hbm, out_sems.at[0])
    pos_copy.start()

    h = _leaky(_bn(_dense(w, roW0, roB0), roG0, roBt0))
    ow_buf[...] = _dense(h, roW1, roB1)
    w_copy = pltpu.make_async_copy(ow_buf, w_out_hbm, out_sems.at[1])
    w_copy.start()

    pos_copy.wait()
    w_copy.wait()


def kernel(positions, weights, params, batch):
    del batch
    n = positions.shape[0]

    flat = []
    for p in params["readin"]:
        flat += [p["W"], p["b"], p["gamma"], p["beta"]]
    for blk in params["blocks"]:
        l0, l1 = blk["delta"]
        flat += [l0["W"], l0["b"], l0["gamma"], l0["beta"], l1["W"], l1["b"]]
    ro0, ro1 = params["readout"]
    flat += [ro0["W"], ro0["b"], ro0["gamma"], ro0["beta"], ro1["W"], ro1["b"]]

    out_dim = ro1["W"].shape[1]
    ins = [weights, positions] + flat
    n_in = len(ins)

    any_spec = pl.BlockSpec(memory_space=pltpu.MemorySpace.HBM)
    pos_out, w_out = pl.pallas_call(
        _forward_kernel,
        in_specs=[any_spec] * n_in,
        out_specs=(any_spec, any_spec),
        out_shape=(
            jax.ShapeDtypeStruct((n, _NDIM), jnp.float32),
            jax.ShapeDtypeStruct((n, out_dim), jnp.float32),
        ),
        scratch_shapes=(
            [pltpu.VMEM(a.shape, a.dtype) for a in ins]
            + [pltpu.VMEM((n, _NDIM), jnp.float32),
               pltpu.VMEM((n, out_dim), jnp.float32),
               pltpu.SemaphoreType.DMA((n_in,)),
               pltpu.SemaphoreType.DMA((2,))]
        ),
    )(*ins)
    return pos_out, w_out
